# baseline (speedup 1.0000x reference)
import functools

import jax
import jax.numpy as jnp
from jax import lax
from jax.experimental import pallas as pl
from jax.experimental.pallas import tpu as pltpu
from jax.experimental.pallas import tpu_sc as plsc

N = 10000
D = 128
E = 320000
NC = 2
NS = 16
NW = NC * NS
EPT = E // NW
K = 128
NSEG = 5
RS = 16
EPT_PAD = NSEG * RS * K
EPAD = NW * EPT_PAD
NBUF = 2
NROW = 10112
TRASH = 10000
ROWS_PT = NROW // NS
NPAD = 10240
DEG_CH = NPAD // NS

_SC_MESH = plsc.VectorSubcoreMesh(core_axis_name="c", subcore_axis_name="s")


@functools.partial(
    pl.kernel,
    out_type=jax.ShapeDtypeStruct((NC, NPAD), jnp.float32),
    mesh=_SC_MESH,
    scratch_types=[
        pltpu.VMEM((RS, K), jnp.int32),
        pltpu.VMEM((K,), jnp.float32),
        pltpu.VMEM((DEG_CH,), jnp.float32),
        pltpu.SemaphoreType.DMA,
        pltpu.VMEM_SHARED((NPAD,), jnp.float32),
    ],
)
def _deg_kernel(dst_hbm, out_hbm, dst_v, ones_v, zeros_v, ssem, deg_sh):
    cid = lax.axis_index("c")
    sid = lax.axis_index("s")

    for i in range(K // 16):
        ones_v[pl.ds(i * 16, 16)] = jnp.ones((16,), jnp.float32)

    def _zb(i, carry):
        zeros_v[pl.ds(i * 16, 16)] = jnp.zeros((16,), jnp.float32)
        return carry

    lax.fori_loop(0, DEG_CH // 16, _zb, 0)
    pltpu.sync_copy(zeros_v, deg_sh.at[pl.ds(sid * DEG_CH, DEG_CH)])
    plsc.subcore_barrier()

    def _fire(j, carry):
        pltpu.async_copy(ones_v, deg_sh.at[dst_v.at[j]], ssem, add=True)
        return carry

    def _drain(j, carry):
        pltpu.make_async_copy(ones_v, deg_sh.at[dst_v.at[0]], ssem).wait()
        return carry

    for seg in range(NSEG):
        pltpu.sync_copy(dst_hbm.at[cid, sid, seg], dst_v)
        lax.fori_loop(0, RS, _fire, 0)
        lax.fori_loop(0, RS, _drain, 0)
    plsc.subcore_barrier()
    pltpu.sync_copy(
        deg_sh.at[pl.ds(sid * DEG_CH, DEG_CH)],
        out_hbm.at[cid, pl.ds(sid * DEG_CH, DEG_CH)],
    )


@functools.partial(
    pl.kernel,
    out_type=jax.ShapeDtypeStruct((NC, NROW, D), jnp.float32),
    mesh=_SC_MESH,
    scratch_types=[
        pltpu.VMEM((RS, K), jnp.int32),
        pltpu.VMEM((RS, K), jnp.int32),
        pltpu.VMEM((K, D), jnp.float32),
        pltpu.VMEM((K, D), jnp.float32),
        pltpu.SemaphoreType.DMA,
        pltpu.SemaphoreType.DMA,
        pltpu.SemaphoreType.DMA,
        pltpu.SemaphoreType.DMA,
        pltpu.VMEM_SHARED((NROW, D), jnp.float32),
    ],
)
def _edge_kernel(src_hbm, dst_hbm, g_hbm, out_hbm,
                 src_v, dst_v, rows0, rows1,
                 gsem0, gsem1, ssem0, ssem1, acc_sh):
    cid = lax.axis_index("c")
    sid = lax.axis_index("s")

    def _zb(i, carry):
        rows0[i // 8, pl.ds((i % 8) * 16, 16)] = jnp.zeros((16,), jnp.float32)
        return carry

    lax.fori_loop(0, K * (D // 16), _zb, 0)
    for c in range(4):
        pltpu.sync_copy(rows0, acc_sh.at[pl.ds(sid * ROWS_PT + c * K, K)])
    pltpu.sync_copy(rows0.at[pl.ds(0, ROWS_PT - 4 * K)],
                    acc_sh.at[pl.ds(sid * ROWS_PT + 4 * K, ROWS_PT - 4 * K)])
    plsc.subcore_barrier()

    rows = (rows0, rows1)
    gsems = (gsem0, gsem1)
    ssems = (ssem0, ssem1)
    for seg in range(NSEG):
        pltpu.sync_copy(src_hbm.at[cid, sid, seg], src_v)
        pltpu.sync_copy(dst_hbm.at[cid, sid, seg], dst_v)
        for b in range(NBUF):
            pltpu.async_copy(g_hbm.at[src_v.at[b]], rows[b], gsems[b])

        def _grp(gi, carry):
            j = gi * NBUF
            for b in range(NBUF):
                jb = j + b
                pltpu.make_async_copy(
                    g_hbm.at[src_v.at[jb]], rows[b], gsems[b]).wait()
                pltpu.async_copy(
                    rows[b], acc_sh.at[dst_v.at[jb]], ssems[b], add=True)
            for b in range(NBUF):
                jb = j + b
                pltpu.make_async_copy(
                    rows[b], acc_sh.at[dst_v.at[jb]], ssems[b]).wait()

                @pl.when(jb + NBUF < RS)
                def _():
                    pltpu.async_copy(
                        g_hbm.at[src_v.at[jb + NBUF]], rows[b], gsems[b])
            return carry

        lax.fori_loop(0, RS // NBUF, _grp, 0)
    plsc.subcore_barrier()
    for c in range(4):
        off = sid * ROWS_PT + c * K
        pltpu.sync_copy(acc_sh.at[pl.ds(off, K)],
                        out_hbm.at[cid, pl.ds(off, K)])
    off = sid * ROWS_PT + 4 * K
    tail = ROWS_PT - 4 * K
    pltpu.sync_copy(acc_sh.at[pl.ds(off, tail)],
                    out_hbm.at[cid, pl.ds(off, tail)])


def _mmscale_body(degt_ref, x_ref, w_ref, g_ref, dinv_ref):
    deg = degt_ref[:, 0:1] + degt_ref[:, 1:2] + 1.0
    dinv = lax.rsqrt(deg)
    dinv_ref[...] = dinv
    h = lax.dot_general(
        x_ref[...], w_ref[...], (((1,), (1,)), ((), ())),
        preferred_element_type=jnp.float32)
    g_ref[...] = dinv * h


def _final_body(acc_ref, g_ref, dinv_ref, b_ref, o_ref):
    o_ref[...] = dinv_ref[...] * (acc_ref[0] + acc_ref[1] + g_ref[...]) \
        + b_ref[...]


_RB = 1000
_GRID = N // _RB


def kernel(x, edge_index, W, b):
    npad = EPAD - E
    src = jnp.concatenate(
        [edge_index[0], jnp.zeros((npad,), jnp.int32)]
    ).reshape(NC, NS, NSEG, RS, K)
    dst = jnp.concatenate(
        [edge_index[1], jnp.full((npad,), TRASH, jnp.int32)]
    ).reshape(NC, NS, NSEG, RS, K)

    degp = _deg_kernel(dst)
    degt = degp[:, :N].T

    g, dinv = pl.pallas_call(
        _mmscale_body,
        grid=(_GRID,),
        in_specs=[
            pl.BlockSpec((_RB, 2), lambda i: (i, 0)),
            pl.BlockSpec((_RB, D), lambda i: (i, 0)),
            pl.BlockSpec((D, D), lambda i: (0, 0)),
        ],
        out_specs=[
            pl.BlockSpec((_RB, D), lambda i: (i, 0)),
            pl.BlockSpec((_RB, 1), lambda i: (i, 0)),
        ],
        out_shape=[
            jax.ShapeDtypeStruct((N, D), jnp.float32),
            jax.ShapeDtypeStruct((N, 1), jnp.float32),
        ],
    )(degt, x, W)

    acc = _edge_kernel(src, dst, g)

    out = pl.pallas_call(
        _final_body,
        grid=(_GRID,),
        in_specs=[
            pl.BlockSpec((NC, _RB, D), lambda i: (0, i, 0)),
            pl.BlockSpec((_RB, D), lambda i: (i, 0)),
            pl.BlockSpec((_RB, 1), lambda i: (i, 0)),
            pl.BlockSpec((1, D), lambda i: (0, 0)),
        ],
        out_specs=pl.BlockSpec((_RB, D), lambda i: (i, 0)),
        out_shape=jax.ShapeDtypeStruct((N, D), jnp.float32),
    )(acc, g, dinv, b.reshape(1, D))

    return out


# speedup vs baseline: 1.0105x; 1.0105x over previous
import functools

import jax
import jax.numpy as jnp
from jax import lax
from jax.experimental import pallas as pl
from jax.experimental.pallas import tpu as pltpu
from jax.experimental.pallas import tpu_sc as plsc

N = 10000
D = 128
E = 320000
NC = 2
NS = 16
NW = NC * NS
EPT = E // NW
K = 128
NSEG = 5
RS = 16
EPT_PAD = NSEG * RS * K
EPAD = NW * EPT_PAD
NBUF = 2
NROW = 10112
TRASH = 10000
ROWS_PT = NROW // NS
NPAD = 10240
DEG_CH = NPAD // NS

_SC_MESH = plsc.VectorSubcoreMesh(core_axis_name="c", subcore_axis_name="s")


@functools.partial(
    pl.kernel,
    out_type=jax.ShapeDtypeStruct((NC, NPAD), jnp.float32),
    mesh=_SC_MESH,
    scratch_types=[
        pltpu.VMEM((RS, K), jnp.int32),
        pltpu.VMEM((K,), jnp.float32),
        pltpu.VMEM((DEG_CH,), jnp.float32),
        pltpu.SemaphoreType.DMA,
        pltpu.VMEM_SHARED((NPAD,), jnp.float32),
    ],
)
def _deg_kernel(dst_hbm, out_hbm, dst_v, ones_v, zeros_v, ssem, deg_sh):
    cid = lax.axis_index("c")
    sid = lax.axis_index("s")

    for i in range(K // 16):
        ones_v[pl.ds(i * 16, 16)] = jnp.ones((16,), jnp.float32)

    def _zb(i, carry):
        zeros_v[pl.ds(i * 16, 16)] = jnp.zeros((16,), jnp.float32)
        return carry

    lax.fori_loop(0, DEG_CH // 16, _zb, 0)
    pltpu.sync_copy(zeros_v, deg_sh.at[pl.ds(sid * DEG_CH, DEG_CH)])
    plsc.subcore_barrier()

    def _fire(j, carry):
        pltpu.async_copy(ones_v, deg_sh.at[dst_v.at[j]], ssem, add=True)
        return carry

    def _drain(j, carry):
        pltpu.make_async_copy(ones_v, deg_sh.at[dst_v.at[0]], ssem).wait()
        return carry

    for seg in range(NSEG):
        pltpu.sync_copy(dst_hbm.at[cid, sid, seg], dst_v)
        lax.fori_loop(0, RS, _fire, 0)
        lax.fori_loop(0, RS, _drain, 0)
    plsc.subcore_barrier()
    pltpu.sync_copy(
        deg_sh.at[pl.ds(sid * DEG_CH, DEG_CH)],
        out_hbm.at[cid, pl.ds(sid * DEG_CH, DEG_CH)],
    )


@functools.partial(
    pl.kernel,
    out_type=jax.ShapeDtypeStruct((NC, NROW, D), jnp.float32),
    mesh=_SC_MESH,
    scratch_types=[
        pltpu.VMEM((RS, K), jnp.int32),
        pltpu.VMEM((RS, K), jnp.int32),
        pltpu.VMEM((K, D), jnp.float32),
        pltpu.VMEM((K, D), jnp.float32),
        pltpu.SemaphoreType.DMA,
        pltpu.SemaphoreType.DMA,
        pltpu.SemaphoreType.DMA,
        pltpu.SemaphoreType.DMA,
        pltpu.VMEM_SHARED((NROW, D), jnp.float32),
    ],
)
def _edge_kernel(src_hbm, dst_hbm, g_hbm, out_hbm,
                 src_v, dst_v, rows0, rows1,
                 gsem0, gsem1, ssem0, ssem1, acc_sh):
    cid = lax.axis_index("c")
    sid = lax.axis_index("s")

    def _zb(i, carry):
        rows0[i // 8, pl.ds((i % 8) * 16, 16)] = jnp.zeros((16,), jnp.float32)
        return carry

    lax.fori_loop(0, K * (D // 16), _zb, 0)
    for c in range(4):
        pltpu.sync_copy(rows0, acc_sh.at[pl.ds(sid * ROWS_PT + c * K, K)])
    pltpu.sync_copy(rows0.at[pl.ds(0, ROWS_PT - 4 * K)],
                    acc_sh.at[pl.ds(sid * ROWS_PT + 4 * K, ROWS_PT - 4 * K)])
    plsc.subcore_barrier()

    rows = (rows0, rows1)
    gsems = (gsem0, gsem1)
    ssems = (ssem0, ssem1)
    for seg in range(NSEG):
        pltpu.sync_copy(src_hbm.at[cid, sid, seg], src_v)
        pltpu.sync_copy(dst_hbm.at[cid, sid, seg], dst_v)
        for b in range(NBUF):
            pltpu.async_copy(g_hbm.at[src_v.at[b]], rows[b], gsems[b])

        def _grp(gi, carry):
            j = gi * NBUF
            for b in range(NBUF):
                jb = j + b
                pltpu.make_async_copy(
                    g_hbm.at[src_v.at[jb]], rows[b], gsems[b]).wait()
                pltpu.async_copy(
                    rows[b], acc_sh.at[dst_v.at[jb]], ssems[b], add=True)
            for b in range(NBUF):
                jb = j + b
                pltpu.make_async_copy(
                    rows[b], acc_sh.at[dst_v.at[jb]], ssems[b]).wait()

                @pl.when(jb + NBUF < RS)
                def _():
                    pltpu.async_copy(
                        g_hbm.at[src_v.at[jb + NBUF]], rows[b], gsems[b])
            return carry

        lax.fori_loop(0, RS // NBUF, _grp, 0)
    plsc.subcore_barrier()
    for c in range(4):
        off = sid * ROWS_PT + c * K
        pltpu.sync_copy(acc_sh.at[pl.ds(off, K)],
                        out_hbm.at[cid, pl.ds(off, K)])
    off = sid * ROWS_PT + 4 * K
    tail = ROWS_PT - 4 * K
    pltpu.sync_copy(acc_sh.at[pl.ds(off, tail)],
                    out_hbm.at[cid, pl.ds(off, tail)])


def _mmscale_body(degt_ref, x_ref, w_ref, g_ref, dinv_ref):
    deg = degt_ref[:, 0:1] + degt_ref[:, 1:2] + 1.0
    dinv = lax.rsqrt(deg)
    dinv_ref[...] = dinv
    h = lax.dot_general(
        x_ref[...], w_ref[...], (((1,), (1,)), ((), ())),
        preferred_element_type=jnp.float32)
    g_ref[...] = dinv * h


def _final_body(acc_ref, g_ref, dinv_ref, b_ref, o_ref):
    o_ref[...] = dinv_ref[...] * (acc_ref[0] + acc_ref[1] + g_ref[...]) \
        + b_ref[...]


_RB = 1000
_GRID = N // _RB


def kernel(x, edge_index, W, b):
    npad = EPAD - E
    src = jnp.concatenate(
        [edge_index[0], jnp.zeros((npad,), jnp.int32)]
    ).reshape(NC, NS, NSEG, RS, K)
    trash = TRASH + jnp.arange(npad, dtype=jnp.int32) % (NROW - TRASH)
    dst = jnp.concatenate(
        [edge_index[1], trash]
    ).reshape(NC, NS, NSEG, RS, K)

    degp = _deg_kernel(dst)
    degt = degp[:, :N].T

    g, dinv = pl.pallas_call(
        _mmscale_body,
        grid=(_GRID,),
        in_specs=[
            pl.BlockSpec((_RB, 2), lambda i: (i, 0)),
            pl.BlockSpec((_RB, D), lambda i: (i, 0)),
            pl.BlockSpec((D, D), lambda i: (0, 0)),
        ],
        out_specs=[
            pl.BlockSpec((_RB, D), lambda i: (i, 0)),
            pl.BlockSpec((_RB, 1), lambda i: (i, 0)),
        ],
        out_shape=[
            jax.ShapeDtypeStruct((N, D), jnp.float32),
            jax.ShapeDtypeStruct((N, 1), jnp.float32),
        ],
    )(degt, x, W)

    acc = _edge_kernel(src, dst, g)

    out = pl.pallas_call(
        _final_body,
        grid=(_GRID,),
        in_specs=[
            pl.BlockSpec((NC, _RB, D), lambda i: (0, i, 0)),
            pl.BlockSpec((_RB, D), lambda i: (i, 0)),
            pl.BlockSpec((_RB, 1), lambda i: (i, 0)),
            pl.BlockSpec((1, D), lambda i: (0, 0)),
        ],
        out_specs=pl.BlockSpec((_RB, D), lambda i: (i, 0)),
        out_shape=jax.ShapeDtypeStruct((N, D), jnp.float32),
    )(acc, g, dinv, b.reshape(1, D))

    return out


# speedup vs baseline: 2.4729x; 2.4473x over previous
import functools

import jax
import jax.numpy as jnp
from jax import lax
from jax.experimental import pallas as pl
from jax.experimental.pallas import tpu as pltpu
from jax.experimental.pallas import tpu_sc as plsc

N = 10000
D = 128
E = 320000
NC = 2
NS = 16
NW = NC * NS
EPT = E // NW
K = 80
NSEG = 5
RS = (EPT // K) // NSEG
NROW = 10240
ROWS_PT = NROW // NS
ZCH = 128
NPAD = 10240
DEG_CH = NPAD // NS

_SC_MESH = plsc.VectorSubcoreMesh(core_axis_name="c", subcore_axis_name="s")


@functools.partial(
    pl.kernel,
    out_type=jax.ShapeDtypeStruct((NC, NPAD), jnp.float32),
    mesh=_SC_MESH,
    scratch_types=[
        pltpu.VMEM((RS, K), jnp.int32),
        pltpu.VMEM((K,), jnp.float32),
        pltpu.VMEM((DEG_CH,), jnp.float32),
        pltpu.SemaphoreType.DMA,
        pltpu.VMEM_SHARED((NPAD,), jnp.float32),
    ],
)
def _deg_kernel(dst_hbm, out_hbm, dst_v, ones_v, zeros_v, ssem, deg_sh):
    cid = lax.axis_index("c")
    sid = lax.axis_index("s")

    for i in range(K // 16):
        ones_v[pl.ds(i * 16, 16)] = jnp.ones((16,), jnp.float32)

    def _zb(i, carry):
        zeros_v[pl.ds(i * 16, 16)] = jnp.zeros((16,), jnp.float32)
        return carry

    lax.fori_loop(0, DEG_CH // 16, _zb, 0)
    pltpu.sync_copy(zeros_v, deg_sh.at[pl.ds(sid * DEG_CH, DEG_CH)])
    plsc.subcore_barrier()

    def _fire(j, carry):
        pltpu.async_copy(ones_v, deg_sh.at[dst_v.at[j]], ssem, add=True)
        return carry

    def _drain(j, carry):
        pltpu.make_async_copy(ones_v, deg_sh.at[dst_v.at[0]], ssem).wait()
        return carry

    for seg in range(NSEG):
        pltpu.sync_copy(dst_hbm.at[cid, sid, seg], dst_v)
        lax.fori_loop(0, RS, _fire, 0)
        lax.fori_loop(0, RS, _drain, 0)
    plsc.subcore_barrier()
    pltpu.sync_copy(
        deg_sh.at[pl.ds(sid * DEG_CH, DEG_CH)],
        out_hbm.at[cid, pl.ds(sid * DEG_CH, DEG_CH)],
    )


@functools.partial(
    pl.kernel,
    out_type=jax.ShapeDtypeStruct((NC, NROW, D), jnp.float32),
    mesh=_SC_MESH,
    scratch_types=[
        pltpu.VMEM((RS, K), jnp.int32),
        pltpu.VMEM((RS, K), jnp.int32),
        pltpu.VMEM((K, D), jnp.float32),
        pltpu.VMEM((K, D), jnp.float32),
        pltpu.SemaphoreType.DMA,
        pltpu.SemaphoreType.DMA,
        pltpu.VMEM_SHARED((NROW, D), jnp.float32),
    ],
)
def _edge_kernel(src_hbm, dst_hbm, g_hbm, out_hbm,
                 src_v, dst_v, rows0, rows1, sem0, sem1, acc_sh):
    cid = lax.axis_index("c")
    sid = lax.axis_index("s")

    def _zb(i, carry):
        rows0[i // 8, pl.ds((i % 8) * 16, 16)] = jnp.zeros((16,), jnp.float32)
        return carry

    lax.fori_loop(0, K * (D // 16), _zb, 0)
    for c in range(ROWS_PT // K):
        pltpu.sync_copy(rows0, acc_sh.at[pl.ds(sid * ROWS_PT + c * K, K)])
    plsc.subcore_barrier()

    rows = (rows0, rows1)
    sems = (sem0, sem1)
    for seg in range(NSEG):
        pltpu.sync_copy(src_hbm.at[cid, sid, seg], src_v)
        pltpu.sync_copy(dst_hbm.at[cid, sid, seg], dst_v)
        pltpu.async_copy(g_hbm.at[src_v.at[0]], rows0, sem0)
        pltpu.async_copy(g_hbm.at[src_v.at[1]], rows1, sem1)

        def _pair(j2, carry):
            j = j2 * 2
            for b in range(2):
                jb = j + b
                pltpu.make_async_copy(
                    g_hbm.at[src_v.at[jb]], rows[b], sems[b]).wait()
                pltpu.sync_copy(rows[b], acc_sh.at[dst_v.at[jb]], add=True)

                @pl.when(jb + 2 < RS)
                def _():
                    pltpu.async_copy(
                        g_hbm.at[src_v.at[jb + 2]], rows[b], sems[b])
            return carry

        lax.fori_loop(0, RS // 2, _pair, 0)
        pltpu.make_async_copy(
            g_hbm.at[src_v.at[RS - 1]], rows0, sem0).wait()
        pltpu.sync_copy(rows0, acc_sh.at[dst_v.at[RS - 1]], add=True)
    plsc.subcore_barrier()
    for c in range(ROWS_PT // ZCH):
        off = sid * ROWS_PT + c * ZCH
        pltpu.sync_copy(acc_sh.at[pl.ds(off, ZCH)],
                        out_hbm.at[cid, pl.ds(off, ZCH)])


def _mmscale_body(degt_ref, x_ref, w_ref, g_ref, dinv_ref):
    deg = degt_ref[:, 0:1] + degt_ref[:, 1:2] + 1.0
    dinv = lax.rsqrt(deg)
    dinv_ref[...] = dinv
    h = lax.dot_general(
        x_ref[...], w_ref[...], (((1,), (1,)), ((), ())),
        preferred_element_type=jnp.float32)
    g_ref[...] = dinv * h


def _final_body(acc_ref, g_ref, dinv_ref, b_ref, o_ref):
    o_ref[...] = dinv_ref[...] * (acc_ref[0] + acc_ref[1] + g_ref[...]) \
        + b_ref[...]


_RB = 1000
_GRID = N // _RB


def kernel(x, edge_index, W, b):
    src = edge_index[0].reshape(NC, NS, NSEG, RS, K)
    dst = edge_index[1].reshape(NC, NS, NSEG, RS, K)

    degp = _deg_kernel(dst)
    degt = degp[:, :N].T

    g, dinv = pl.pallas_call(
        _mmscale_body,
        grid=(_GRID,),
        in_specs=[
            pl.BlockSpec((_RB, 2), lambda i: (i, 0)),
            pl.BlockSpec((_RB, D), lambda i: (i, 0)),
            pl.BlockSpec((D, D), lambda i: (0, 0)),
        ],
        out_specs=[
            pl.BlockSpec((_RB, D), lambda i: (i, 0)),
            pl.BlockSpec((_RB, 1), lambda i: (i, 0)),
        ],
        out_shape=[
            jax.ShapeDtypeStruct((N, D), jnp.float32),
            jax.ShapeDtypeStruct((N, 1), jnp.float32),
        ],
    )(degt, x, W)

    acc = _edge_kernel(src, dst, g)

    out = pl.pallas_call(
        _final_body,
        grid=(_GRID,),
        in_specs=[
            pl.BlockSpec((NC, _RB, D), lambda i: (0, i, 0)),
            pl.BlockSpec((_RB, D), lambda i: (i, 0)),
            pl.BlockSpec((_RB, 1), lambda i: (i, 0)),
            pl.BlockSpec((1, D), lambda i: (0, 0)),
        ],
        out_specs=pl.BlockSpec((_RB, D), lambda i: (i, 0)),
        out_shape=jax.ShapeDtypeStruct((N, D), jnp.float32),
    )(acc, g, dinv, b.reshape(1, D))

    return out


# speedup vs baseline: 2.5462x; 1.0297x over previous
import functools

import jax
import jax.numpy as jnp
from jax import lax
from jax.experimental import pallas as pl
from jax.experimental.pallas import tpu as pltpu
from jax.experimental.pallas import tpu_sc as plsc

N = 10000
D = 128
E = 320000
NC = 2
NS = 16
NW = NC * NS
EPT = E // NW
K = 100
NSEG = 5
RS = (EPT // K) // NSEG
NROW = 10240
ROWS_PT = NROW // NS
ZCH = 128
NPAD = 10240
DEG_CH = NPAD // NS

_SC_MESH = plsc.VectorSubcoreMesh(core_axis_name="c", subcore_axis_name="s")


@functools.partial(
    pl.kernel,
    out_type=jax.ShapeDtypeStruct((NC, NPAD), jnp.float32),
    mesh=_SC_MESH,
    scratch_types=[
        pltpu.VMEM((RS, K), jnp.int32),
        pltpu.VMEM((K,), jnp.float32),
        pltpu.VMEM((DEG_CH,), jnp.float32),
        pltpu.SemaphoreType.DMA,
        pltpu.VMEM_SHARED((NPAD,), jnp.float32),
    ],
)
def _deg_kernel(dst_hbm, out_hbm, dst_v, ones_v, zeros_v, ssem, deg_sh):
    cid = lax.axis_index("c")
    sid = lax.axis_index("s")

    for i in range(K // 16):
        ones_v[pl.ds(i * 16, 16)] = jnp.ones((16,), jnp.float32)

    def _zb(i, carry):
        zeros_v[pl.ds(i * 16, 16)] = jnp.zeros((16,), jnp.float32)
        return carry

    lax.fori_loop(0, DEG_CH // 16, _zb, 0)
    pltpu.sync_copy(zeros_v, deg_sh.at[pl.ds(sid * DEG_CH, DEG_CH)])
    plsc.subcore_barrier()

    def _fire(j, carry):
        pltpu.async_copy(ones_v, deg_sh.at[dst_v.at[j]], ssem, add=True)
        return carry

    def _drain(j, carry):
        pltpu.make_async_copy(ones_v, deg_sh.at[dst_v.at[0]], ssem).wait()
        return carry

    for seg in range(NSEG):
        pltpu.sync_copy(dst_hbm.at[cid, sid, seg], dst_v)
        lax.fori_loop(0, RS, _fire, 0)
        lax.fori_loop(0, RS, _drain, 0)
    plsc.subcore_barrier()
    pltpu.sync_copy(
        deg_sh.at[pl.ds(sid * DEG_CH, DEG_CH)],
        out_hbm.at[cid, pl.ds(sid * DEG_CH, DEG_CH)],
    )


@functools.partial(
    pl.kernel,
    out_type=jax.ShapeDtypeStruct((NC, NROW, D), jnp.float32),
    mesh=_SC_MESH,
    scratch_types=[
        pltpu.VMEM((RS, K), jnp.int32),
        pltpu.VMEM((RS, K), jnp.int32),
        pltpu.VMEM((K, D), jnp.float32),
        pltpu.VMEM((K, D), jnp.float32),
        pltpu.SemaphoreType.DMA,
        pltpu.SemaphoreType.DMA,
        pltpu.VMEM_SHARED((NROW, D), jnp.float32),
    ],
)
def _edge_kernel(src_hbm, dst_hbm, g_hbm, out_hbm,
                 src_v, dst_v, rows0, rows1, sem0, sem1, acc_sh):
    cid = lax.axis_index("c")
    sid = lax.axis_index("s")

    def _zb(i, carry):
        rows0[i // 8, pl.ds((i % 8) * 16, 16)] = jnp.zeros((16,), jnp.float32)
        return carry

    lax.fori_loop(0, K * (D // 16), _zb, 0)
    for c in range(ROWS_PT // 80):
        pltpu.sync_copy(rows0.at[pl.ds(0, 80)],
                        acc_sh.at[pl.ds(sid * ROWS_PT + c * 80, 80)])
    plsc.subcore_barrier()

    rows = (rows0, rows1)
    sems = (sem0, sem1)
    for seg in range(NSEG):
        pltpu.sync_copy(src_hbm.at[cid, sid, seg], src_v)
        pltpu.sync_copy(dst_hbm.at[cid, sid, seg], dst_v)
        pltpu.async_copy(g_hbm.at[src_v.at[0]], rows0, sem0)
        pltpu.async_copy(g_hbm.at[src_v.at[1]], rows1, sem1)

        def _pair(j2, carry):
            j = j2 * 2
            for b in range(2):
                jb = j + b
                pltpu.make_async_copy(
                    g_hbm.at[src_v.at[jb]], rows[b], sems[b]).wait()
                pltpu.sync_copy(rows[b], acc_sh.at[dst_v.at[jb]], add=True)

                @pl.when(jb + 2 < RS)
                def _():
                    pltpu.async_copy(
                        g_hbm.at[src_v.at[jb + 2]], rows[b], sems[b])
            return carry

        lax.fori_loop(0, RS // 2, _pair, 0)
        if RS % 2:
            pltpu.make_async_copy(
                g_hbm.at[src_v.at[RS - 1]], rows0, sem0).wait()
            pltpu.sync_copy(rows0, acc_sh.at[dst_v.at[RS - 1]], add=True)
    plsc.subcore_barrier()
    for c in range(ROWS_PT // ZCH):
        off = sid * ROWS_PT + c * ZCH
        pltpu.sync_copy(acc_sh.at[pl.ds(off, ZCH)],
                        out_hbm.at[cid, pl.ds(off, ZCH)])


def _mmscale_body(degt_ref, x_ref, w_ref, g_ref, dinv_ref):
    deg = degt_ref[:, 0:1] + degt_ref[:, 1:2] + 1.0
    dinv = lax.rsqrt(deg)
    dinv_ref[...] = dinv
    h = lax.dot_general(
        x_ref[...], w_ref[...], (((1,), (1,)), ((), ())),
        preferred_element_type=jnp.float32)
    g_ref[...] = dinv * h


def _final_body(acc_ref, g_ref, dinv_ref, b_ref, o_ref):
    o_ref[...] = dinv_ref[...] * (acc_ref[0] + acc_ref[1] + g_ref[...]) \
        + b_ref[...]


_RB = 1000
_GRID = N // _RB


def kernel(x, edge_index, W, b):
    src = edge_index[0].reshape(NC, NS, NSEG, RS, K)
    dst = edge_index[1].reshape(NC, NS, NSEG, RS, K)

    degp = _deg_kernel(dst)
    degt = degp[:, :N].T

    g, dinv = pl.pallas_call(
        _mmscale_body,
        grid=(_GRID,),
        in_specs=[
            pl.BlockSpec((_RB, 2), lambda i: (i, 0)),
            pl.BlockSpec((_RB, D), lambda i: (i, 0)),
            pl.BlockSpec((D, D), lambda i: (0, 0)),
        ],
        out_specs=[
            pl.BlockSpec((_RB, D), lambda i: (i, 0)),
            pl.BlockSpec((_RB, 1), lambda i: (i, 0)),
        ],
        out_shape=[
            jax.ShapeDtypeStruct((N, D), jnp.float32),
            jax.ShapeDtypeStruct((N, 1), jnp.float32),
        ],
    )(degt, x, W)

    acc = _edge_kernel(src, dst, g)

    out = pl.pallas_call(
        _final_body,
        grid=(_GRID,),
        in_specs=[
            pl.BlockSpec((NC, _RB, D), lambda i: (0, i, 0)),
            pl.BlockSpec((_RB, D), lambda i: (i, 0)),
            pl.BlockSpec((_RB, 1), lambda i: (i, 0)),
            pl.BlockSpec((1, D), lambda i: (0, 0)),
        ],
        out_specs=pl.BlockSpec((_RB, D), lambda i: (i, 0)),
        out_shape=jax.ShapeDtypeStruct((N, D), jnp.float32),
    )(acc, g, dinv, b.reshape(1, D))

    return out


# speedup vs baseline: 2.5821x; 1.0141x over previous
import functools

import jax
import jax.numpy as jnp
from jax import lax
from jax.experimental import pallas as pl
from jax.experimental.pallas import tpu as pltpu
from jax.experimental.pallas import tpu_sc as plsc

N = 10000
D = 128
E = 320000
NC = 2
NS = 16
NW = NC * NS
EPT = E // NW
K = 125
NSEG = 5
RS = (EPT // K) // NSEG
NROW = 10240
ROWS_PT = NROW // NS
ZCH = 128
NPAD = 10240
DEG_CH = NPAD // NS

_SC_MESH = plsc.VectorSubcoreMesh(core_axis_name="c", subcore_axis_name="s")


@functools.partial(
    pl.kernel,
    out_type=jax.ShapeDtypeStruct((NC, NPAD), jnp.float32),
    mesh=_SC_MESH,
    scratch_types=[
        pltpu.VMEM((RS, K), jnp.int32),
        pltpu.VMEM((K,), jnp.float32),
        pltpu.VMEM((DEG_CH,), jnp.float32),
        pltpu.SemaphoreType.DMA,
        pltpu.VMEM_SHARED((NPAD,), jnp.float32),
    ],
)
def _deg_kernel(dst_hbm, out_hbm, dst_v, ones_v, zeros_v, ssem, deg_sh):
    cid = lax.axis_index("c")
    sid = lax.axis_index("s")

    for i in range(K // 16):
        ones_v[pl.ds(i * 16, 16)] = jnp.ones((16,), jnp.float32)

    def _zb(i, carry):
        zeros_v[pl.ds(i * 16, 16)] = jnp.zeros((16,), jnp.float32)
        return carry

    lax.fori_loop(0, DEG_CH // 16, _zb, 0)
    pltpu.sync_copy(zeros_v, deg_sh.at[pl.ds(sid * DEG_CH, DEG_CH)])
    plsc.subcore_barrier()

    def _fire(j, carry):
        pltpu.async_copy(ones_v, deg_sh.at[dst_v.at[j]], ssem, add=True)
        return carry

    def _drain(j, carry):
        pltpu.make_async_copy(ones_v, deg_sh.at[dst_v.at[0]], ssem).wait()
        return carry

    for seg in range(NSEG):
        pltpu.sync_copy(dst_hbm.at[cid, sid, seg], dst_v)
        lax.fori_loop(0, RS, _fire, 0)
        lax.fori_loop(0, RS, _drain, 0)
    plsc.subcore_barrier()
    pltpu.sync_copy(
        deg_sh.at[pl.ds(sid * DEG_CH, DEG_CH)],
        out_hbm.at[cid, pl.ds(sid * DEG_CH, DEG_CH)],
    )


@functools.partial(
    pl.kernel,
    out_type=jax.ShapeDtypeStruct((NC, NROW, D), jnp.float32),
    mesh=_SC_MESH,
    scratch_types=[
        pltpu.VMEM((RS, K), jnp.int32),
        pltpu.VMEM((RS, K), jnp.int32),
        pltpu.VMEM((K, D), jnp.float32),
        pltpu.VMEM((K, D), jnp.float32),
        pltpu.SemaphoreType.DMA,
        pltpu.SemaphoreType.DMA,
        pltpu.VMEM_SHARED((NROW, D), jnp.float32),
    ],
)
def _edge_kernel(src_hbm, dst_hbm, g_hbm, out_hbm,
                 src_v, dst_v, rows0, rows1, sem0, sem1, acc_sh):
    cid = lax.axis_index("c")
    sid = lax.axis_index("s")

    def _zb(i, carry):
        rows0[i // 8, pl.ds((i % 8) * 16, 16)] = jnp.zeros((16,), jnp.float32)
        return carry

    lax.fori_loop(0, K * (D // 16), _zb, 0)
    for c in range(ROWS_PT // 80):
        pltpu.sync_copy(rows0.at[pl.ds(0, 80)],
                        acc_sh.at[pl.ds(sid * ROWS_PT + c * 80, 80)])
    plsc.subcore_barrier()

    rows = (rows0, rows1)
    sems = (sem0, sem1)
    for seg in range(NSEG):
        pltpu.sync_copy(src_hbm.at[cid, sid, seg], src_v)
        pltpu.sync_copy(dst_hbm.at[cid, sid, seg], dst_v)
        pltpu.async_copy(g_hbm.at[src_v.at[0]], rows0, sem0)
        pltpu.async_copy(g_hbm.at[src_v.at[1]], rows1, sem1)

        def _pair(j2, carry):
            j = j2 * 2
            for b in range(2):
                jb = j + b
                pltpu.make_async_copy(
                    g_hbm.at[src_v.at[jb]], rows[b], sems[b]).wait()
                pltpu.sync_copy(rows[b], acc_sh.at[dst_v.at[jb]], add=True)

                @pl.when(jb + 2 < RS)
                def _():
                    pltpu.async_copy(
                        g_hbm.at[src_v.at[jb + 2]], rows[b], sems[b])
            return carry

        lax.fori_loop(0, RS // 2, _pair, 0)
        if RS % 2:
            pltpu.make_async_copy(
                g_hbm.at[src_v.at[RS - 1]], rows0, sem0).wait()
            pltpu.sync_copy(rows0, acc_sh.at[dst_v.at[RS - 1]], add=True)
    plsc.subcore_barrier()
    for c in range(ROWS_PT // ZCH):
        off = sid * ROWS_PT + c * ZCH
        pltpu.sync_copy(acc_sh.at[pl.ds(off, ZCH)],
                        out_hbm.at[cid, pl.ds(off, ZCH)])


def _mmscale_body(degt_ref, x_ref, w_ref, g_ref, dinv_ref):
    deg = degt_ref[:, 0:1] + degt_ref[:, 1:2] + 1.0
    dinv = lax.rsqrt(deg)
    dinv_ref[...] = dinv
    h = lax.dot_general(
        x_ref[...], w_ref[...], (((1,), (1,)), ((), ())),
        preferred_element_type=jnp.float32)
    g_ref[...] = dinv * h


def _final_body(acc_ref, g_ref, dinv_ref, b_ref, o_ref):
    o_ref[...] = dinv_ref[...] * (acc_ref[0] + acc_ref[1] + g_ref[...]) \
        + b_ref[...]


_RB = 1000
_GRID = N // _RB


def kernel(x, edge_index, W, b):
    src = edge_index[0].reshape(NC, NS, NSEG, RS, K)
    dst = edge_index[1].reshape(NC, NS, NSEG, RS, K)

    degp = _deg_kernel(dst)
    degt = degp[:, :N].T

    g, dinv = pl.pallas_call(
        _mmscale_body,
        grid=(_GRID,),
        in_specs=[
            pl.BlockSpec((_RB, 2), lambda i: (i, 0)),
            pl.BlockSpec((_RB, D), lambda i: (i, 0)),
            pl.BlockSpec((D, D), lambda i: (0, 0)),
        ],
        out_specs=[
            pl.BlockSpec((_RB, D), lambda i: (i, 0)),
            pl.BlockSpec((_RB, 1), lambda i: (i, 0)),
        ],
        out_shape=[
            jax.ShapeDtypeStruct((N, D), jnp.float32),
            jax.ShapeDtypeStruct((N, 1), jnp.float32),
        ],
    )(degt, x, W)

    acc = _edge_kernel(src, dst, g)

    out = pl.pallas_call(
        _final_body,
        grid=(_GRID,),
        in_specs=[
            pl.BlockSpec((NC, _RB, D), lambda i: (0, i, 0)),
            pl.BlockSpec((_RB, D), lambda i: (i, 0)),
            pl.BlockSpec((_RB, 1), lambda i: (i, 0)),
            pl.BlockSpec((1, D), lambda i: (0, 0)),
        ],
        out_specs=pl.BlockSpec((_RB, D), lambda i: (i, 0)),
        out_shape=jax.ShapeDtypeStruct((N, D), jnp.float32),
    )(acc, g, dinv, b.reshape(1, D))

    return out
